# baseline (speedup 1.0000x reference)
import functools

import jax
import jax.numpy as jnp
from jax import lax
from jax.experimental import pallas as pl
from jax.experimental.pallas import tpu as pltpu
from jax.experimental.pallas import tpu_sc as plsc

_D = 128
_H = 512
_B = 4096
_EPS = 1e-5

_NC = 1
_NS = 16
_NW = _NC * _NS
_BPW = _B // _NW
_NCH = 4
_CH = _BPW // _NCH


def _make_sc_gather():
  mesh = plsc.VectorSubcoreMesh(core_axis_name="c", subcore_axis_name="s",
                                num_cores=1)

  @functools.partial(
      pl.kernel,
      mesh=mesh,
      out_type=jax.ShapeDtypeStruct((_B, _D), jnp.float32),
      scratch_types=(
          [pltpu.VMEM((_CH,), jnp.int32) for _ in range(_NCH)]
          + [pltpu.VMEM((_CH, _D), jnp.float32) for _ in range(_NCH)]
          + [pltpu.SemaphoreType.DMA for _ in range(2 * _NCH)]
      ),
  )
  def gather_kernel(idx_hbm, table_hbm, out_hbm, *scr):
    idxs = scr[:_NCH]
    rows = scr[_NCH:2 * _NCH]
    gsem = scr[2 * _NCH:3 * _NCH]
    wsem = scr[3 * _NCH:]
    wid = lax.axis_index("s") * _NC + lax.axis_index("c")
    base = wid * _BPW
    gs = []
    for k in range(_NCH):
      pltpu.sync_copy(idx_hbm.at[pl.ds(base + k * _CH, _CH)], idxs[k])
      gs.append(pltpu.async_copy(table_hbm.at[idxs[k]], rows[k], gsem[k]))
    ws = []
    for k in range(_NCH):
      gs[k].wait()
      ws.append(pltpu.async_copy(
          rows[k], out_hbm.at[pl.ds(base + k * _CH, _CH)], wsem[k]))
    for w in ws:
      w.wait()

  return gather_kernel


_sc_gather = _make_sc_gather()

_BLK = 2048


def _tc_body(x_ref, w_ref, b_ref, g_ref, bt_ref, o_ref):
  h = jnp.dot(x_ref[...], w_ref[...], preferred_element_type=jnp.float32)
  h = h + b_ref[...]
  mean = jnp.mean(h, axis=-1, keepdims=True)
  var = jnp.mean(jnp.square(h - mean), axis=-1, keepdims=True)
  o_ref[...] = (h - mean) * lax.rsqrt(var + _EPS) * g_ref[...] + bt_ref[...]


def _tc_proj_ln(x, W, b, gamma, beta):
  grid = _B // _BLK
  return pl.pallas_call(
      _tc_body,
      grid=(grid,),
      in_specs=[
          pl.BlockSpec((_BLK, _D), lambda i: (i, 0)),
          pl.BlockSpec((_D, _H), lambda i: (0, 0)),
          pl.BlockSpec((1, _H), lambda i: (0, 0)),
          pl.BlockSpec((1, _H), lambda i: (0, 0)),
          pl.BlockSpec((1, _H), lambda i: (0, 0)),
      ],
      out_specs=pl.BlockSpec((_BLK, _H), lambda i: (i, 0)),
      out_shape=jax.ShapeDtypeStruct((_B, _H), jnp.float32),
  )(x, W, b, gamma, beta)


@jax.jit
def kernel(user_ids, table, W, b, gamma, beta):
  idx = user_ids.astype(jnp.int32)
  embeds = _sc_gather(idx, table)
  return _tc_proj_ln(embeds, W, b.reshape(1, _H), gamma.reshape(1, _H),
                     beta.reshape(1, _H))
